# baseline (speedup 1.0000x reference)
import functools

import jax
import jax.numpy as jnp
from jax import lax
from jax.experimental import pallas as pl
from jax.experimental.pallas import tpu as pltpu
from jax.experimental.pallas import tpu_sc as plsc

B = 16384
L = 50
D = 16
EPS = 1e-3

NC = 2
NS = 16
NW = NC * NS
RPW = B // NW
CH = 32
NCHUNK = RPW // CH

C_ITEMH = 0
C_CATEH = 56
C_SING = 112
LP = 56


def _sc_body(ids_cc, t_u, t_i, t_c, t_s, t_hi, t_hc, gath,
             idxb, sidxb, sbuf, hbuf, obuf, sem):
    wid = lax.axis_index("s") * NC + lax.axis_index("c")
    row0 = wid * RPW
    iota = lax.iota(jnp.int32, 16)
    zeros = jnp.zeros((16,), jnp.float32)
    singles = (t_u, t_i, t_c, t_s)

    def chunk_body(ch, _):
        base = row0 + ch * CH
        pltpu.async_copy(ids_cc.at[pl.ds(base, CH)], idxb, sem).wait()
        for t in range(4):
            for h in range(2):
                v = plsc.load_gather(idxb, [h * 16 + iota,
                                            iota * 0 + (C_SING + t)])
                plsc.store_scatter(sidxb, [iota * 0 + t, h * 16 + iota], v)
        dmas = []
        for j in range(CH):
            dmas.append(pltpu.async_copy(
                t_hi.at[idxb.at[j, pl.ds(C_ITEMH, LP)]],
                hbuf.at[pl.ds(j * LP, LP)], sem))
        for j in range(CH):
            dmas.append(pltpu.async_copy(
                t_hc.at[idxb.at[j, pl.ds(C_CATEH, LP)]],
                hbuf.at[pl.ds(CH * LP + j * LP, LP)], sem))
        for t in range(4):
            dmas.append(pltpu.async_copy(
                singles[t].at[sidxb.at[t]],
                sbuf.at[pl.ds(t * CH, CH)], sem))
        for hnd in dmas:
            hnd.wait()

        def row_body(r, _):
            fr = iota * 0 + r
            for t in range(4):
                v = sbuf[t * CH + r, :]
                plsc.store_scatter(obuf, [fr, t * 16 + iota], v)
            acc_i = zeros
            for j in range(L):
                acc_i = acc_i + hbuf[r * LP + j, :]
            plsc.store_scatter(obuf, [fr, 64 + iota], acc_i * (1.0 / L))
            acc_c = zeros
            for j in range(L):
                acc_c = acc_c + hbuf[CH * LP + r * LP + j, :]
            plsc.store_scatter(obuf, [fr, 80 + iota], acc_c * (1.0 / L))
            return 0

        lax.fori_loop(0, CH, row_body, 0)
        pltpu.sync_copy(obuf, gath.at[pl.ds(base, CH)])
        return 0

    lax.fori_loop(0, NCHUNK, chunk_body, 0)


@functools.partial(
    pl.kernel,
    out_type=jax.ShapeDtypeStruct((B, 128), jnp.float32),
    mesh=plsc.VectorSubcoreMesh(core_axis_name="c", subcore_axis_name="s",
                                num_cores=NC),
    compiler_params=pltpu.CompilerParams(needs_layout_passes=False,
                                         use_tc_tiling_on_sc=False),
    scratch_types=[
        pltpu.VMEM((CH, 128), jnp.int32),
        pltpu.VMEM((4, CH), jnp.int32),
        pltpu.VMEM((4 * CH, D), jnp.float32),
        pltpu.VMEM((2 * CH * LP, D), jnp.float32),
        pltpu.VMEM((CH, 128), jnp.float32),
        pltpu.SemaphoreType.DMA,
    ],
)
def _sc_kernel(*args):
    _sc_body(*args)


RB = 1024


def _tc_body(gath_ref, price_ref, ctr_ref, hp_ref, cst_ref, out_ref):
    ps = cst_ref[0, 0]
    pb = cst_ref[1, 0]
    wce = cst_ref[2:3, :]
    bce = cst_ref[3:4, :]
    whp = cst_ref[4:5, :]
    bhp = cst_ref[5:6, :]
    out_ref[:, 0:1] = price_ref[:, :] * ps + pb
    out_ref[:, 1:17] = ctr_ref[:, :] * wce + bce
    out_ref[:, 17:81] = gath_ref[:, 0:64]
    s = jnp.sum(hp_ref[:, :], axis=1, keepdims=True)
    out_ref[:, 81:97] = s * whp + bhp
    out_ref[:, 97:129] = gath_ref[:, 64:96]


_tc_kernel = pl.pallas_call(
    _tc_body,
    out_shape=jax.ShapeDtypeStruct((B, 129), jnp.float32),
    grid=(B // RB,),
    in_specs=[
        pl.BlockSpec((RB, 128), lambda i: (i, 0)),
        pl.BlockSpec((RB, 1), lambda i: (i, 0)),
        pl.BlockSpec((RB, 1), lambda i: (i, 0)),
        pl.BlockSpec((RB, L), lambda i: (i, 0)),
        pl.BlockSpec((6, 16), lambda i: (0, 0)),
    ],
    out_specs=pl.BlockSpec((RB, 129), lambda i: (i, 0)),
)


def kernel(price, ctr, user_id, item_id, cate_id, shop_id, hist_item_id,
           hist_cate_id, hist_price, T_user, T_item, T_cate, T_shop,
           T_hist_item, T_hist_cate, W_ctr, W_hp,
           g_price, b_price, g_ctr, b_ctr, g_hp, b_hp):
    rs = 1.0 / jnp.sqrt(jnp.float32(1.0 + EPS))
    consts = jnp.stack([
        jnp.broadcast_to(g_price[0] * rs, (16,)),
        jnp.broadcast_to(b_price[0], (16,)),
        (g_ctr[0] * rs) * W_ctr[0],
        b_ctr[0] * W_ctr[0],
        (g_hp[0] * rs / L) * W_hp[0],
        b_hp[0] * W_hp[0],
    ]).astype(jnp.float32)
    zi6 = jnp.zeros((B, 6), jnp.int32)
    ids_cc = jnp.concatenate(
        [hist_item_id, zi6, hist_cate_id, zi6,
         user_id, item_id, cate_id, shop_id,
         jnp.zeros((B, 12), jnp.int32)], axis=1)
    gath = _sc_kernel(ids_cc, T_user, T_item, T_cate, T_shop,
                      T_hist_item, T_hist_cate)
    return _tc_kernel(gath, price, ctr, hist_price, consts)


# speedup vs baseline: 1.2128x; 1.2128x over previous
import functools

import jax
import jax.numpy as jnp
from jax import lax
from jax.experimental import pallas as pl
from jax.experimental.pallas import tpu as pltpu
from jax.experimental.pallas import tpu_sc as plsc

B = 16384
L = 50
D = 16
EPS = 1e-3

NC = 2
NS = 16
NW = NC * NS
RPW = B // NW
CH = 32
NCHUNK = RPW // CH

C_ITEMH = 0
C_CATEH = 56
C_SING = 112
LP = 56


def _sc_hist_body(ids_cc, t_hi, t_hc, gath, idxb, hbuf, obuf, sem):
    wid = lax.axis_index("s") * NC + lax.axis_index("c")
    row0 = wid * RPW
    iota = lax.iota(jnp.int32, 16)
    zeros = jnp.zeros((16,), jnp.float32)

    def chunk_body(ch, _):
        base = row0 + ch * CH
        pltpu.async_copy(ids_cc.at[pl.ds(base, CH)], idxb, sem).wait()
        dmas = []
        for j in range(CH):
            dmas.append(pltpu.async_copy(
                t_hi.at[idxb.at[j, pl.ds(C_ITEMH, LP)]],
                hbuf.at[pl.ds(j * LP, LP)], sem))
        for j in range(CH):
            dmas.append(pltpu.async_copy(
                t_hc.at[idxb.at[j, pl.ds(C_CATEH, LP)]],
                hbuf.at[pl.ds(CH * LP + j * LP, LP)], sem))
        for hnd in dmas:
            hnd.wait()

        def row_body(r, _):
            fr = iota * 0 + r
            acc_i = zeros
            for j in range(L):
                acc_i = acc_i + hbuf[r * LP + j, :]
            plsc.store_scatter(obuf, [fr, iota], acc_i * (1.0 / L))
            acc_c = zeros
            for j in range(L):
                acc_c = acc_c + hbuf[CH * LP + r * LP + j, :]
            plsc.store_scatter(obuf, [fr, 16 + iota], acc_c * (1.0 / L))
            return 0

        lax.fori_loop(0, CH, row_body, 0)
        pltpu.sync_copy(obuf, gath.at[pl.ds(base, CH)])
        return 0

    lax.fori_loop(0, NCHUNK, chunk_body, 0)


@functools.partial(
    pl.kernel,
    out_type=jax.ShapeDtypeStruct((B, 32), jnp.float32),
    mesh=plsc.VectorSubcoreMesh(core_axis_name="c", subcore_axis_name="s",
                                num_cores=NC),
    compiler_params=pltpu.CompilerParams(needs_layout_passes=False,
                                         use_tc_tiling_on_sc=False),
    scratch_types=[
        pltpu.VMEM((CH, 128), jnp.int32),
        pltpu.VMEM((2 * CH * LP, D), jnp.float32),
        pltpu.VMEM((CH, 32), jnp.float32),
        pltpu.SemaphoreType.DMA,
    ],
)
def _sc_hist_kernel(*args):
    _sc_hist_body(*args)


def _sc_single_body(ids_cc, t_u, t_i, t_c, t_s, gath, idxb, sbuf, obuf, sem):
    wid = lax.axis_index("s") * NC + lax.axis_index("c")
    row0 = wid * RPW
    iota = lax.iota(jnp.int32, 16)
    singles = (t_u, t_i, t_c, t_s)

    def chunk_body(ch, _):
        base = row0 + ch * CH
        pltpu.async_copy(ids_cc.at[pl.ds(base, CH)], idxb, sem).wait()
        dmas = []
        for j in range(CH):
            sv = idxb[j, pl.ds(C_SING, 16)]
            for t in range(4):
                idx = sv[t]
                dmas.append(pltpu.async_copy(
                    singles[t].at[pl.ds(idx, 1)],
                    sbuf.at[pl.ds(4 * j + t, 1)], sem))
        for hnd in dmas:
            hnd.wait()

        def row_body(r, _):
            fr = iota * 0 + r
            for t in range(4):
                v = sbuf[4 * r + t, :]
                plsc.store_scatter(obuf, [fr, t * 16 + iota], v)
            return 0

        lax.fori_loop(0, CH, row_body, 0)
        pltpu.sync_copy(obuf, gath.at[pl.ds(base, CH)])
        return 0

    lax.fori_loop(0, NCHUNK, chunk_body, 0)


@functools.partial(
    pl.kernel,
    out_type=jax.ShapeDtypeStruct((B, 128), jnp.float32),
    mesh=plsc.VectorSubcoreMesh(core_axis_name="c", subcore_axis_name="s",
                                num_cores=NC),
    compiler_params=pltpu.CompilerParams(needs_layout_passes=False,
                                         use_tc_tiling_on_sc=True),
    scratch_types=[
        pltpu.VMEM((CH, 128), jnp.int32),
        pltpu.VMEM((4 * CH, D), jnp.float32),
        pltpu.VMEM((CH, 128), jnp.float32),
        pltpu.SemaphoreType.DMA,
    ],
)
def _sc_single_kernel(*args):
    _sc_single_body(*args)


RB = 1024


def _tc_body(sing_ref, hist_ref, price_ref, ctr_ref, hp_ref, cst_ref,
             out_ref):
    ps = cst_ref[0, 0]
    pb = cst_ref[1, 0]
    wce = cst_ref[2:3, :]
    bce = cst_ref[3:4, :]
    whp = cst_ref[4:5, :]
    bhp = cst_ref[5:6, :]
    out_ref[:, 0:1] = price_ref[:, :] * ps + pb
    out_ref[:, 1:17] = ctr_ref[:, :] * wce + bce
    out_ref[:, 17:81] = sing_ref[:, 0:64]
    s = jnp.sum(hp_ref[:, :], axis=1, keepdims=True)
    out_ref[:, 81:97] = s * whp + bhp
    out_ref[:, 97:129] = hist_ref[:, :]


_tc_kernel = pl.pallas_call(
    _tc_body,
    out_shape=jax.ShapeDtypeStruct((B, 129), jnp.float32),
    grid=(B // RB,),
    in_specs=[
        pl.BlockSpec((RB, 128), lambda i: (i, 0)),
        pl.BlockSpec((RB, 32), lambda i: (i, 0)),
        pl.BlockSpec((RB, 1), lambda i: (i, 0)),
        pl.BlockSpec((RB, 1), lambda i: (i, 0)),
        pl.BlockSpec((RB, L), lambda i: (i, 0)),
        pl.BlockSpec((6, 16), lambda i: (0, 0)),
    ],
    out_specs=pl.BlockSpec((RB, 129), lambda i: (i, 0)),
)


def kernel(price, ctr, user_id, item_id, cate_id, shop_id, hist_item_id,
           hist_cate_id, hist_price, T_user, T_item, T_cate, T_shop,
           T_hist_item, T_hist_cate, W_ctr, W_hp,
           g_price, b_price, g_ctr, b_ctr, g_hp, b_hp):
    rs = 1.0 / jnp.sqrt(jnp.float32(1.0 + EPS))
    consts = jnp.stack([
        jnp.broadcast_to(g_price[0] * rs, (16,)),
        jnp.broadcast_to(b_price[0], (16,)),
        (g_ctr[0] * rs) * W_ctr[0],
        b_ctr[0] * W_ctr[0],
        (g_hp[0] * rs / L) * W_hp[0],
        b_hp[0] * W_hp[0],
    ]).astype(jnp.float32)
    zi6 = jnp.zeros((B, 6), jnp.int32)
    ids_cc = jnp.concatenate(
        [hist_item_id, zi6, hist_cate_id, zi6,
         user_id, item_id, cate_id, shop_id,
         jnp.zeros((B, 12), jnp.int32)], axis=1)
    sing = _sc_single_kernel(ids_cc, T_user, T_item, T_cate, T_shop)
    hist = _sc_hist_kernel(ids_cc, T_hist_item, T_hist_cate)
    return _tc_kernel(sing, hist, price, ctr, hist_price, consts)


# speedup vs baseline: 1.3525x; 1.1152x over previous
import functools

import jax
import jax.numpy as jnp
from jax import lax
from jax.experimental import pallas as pl
from jax.experimental.pallas import tpu as pltpu
from jax.experimental.pallas import tpu_sc as plsc

B = 16384
L = 50
D = 16
EPS = 1e-3

NC = 2
NS = 16
NW = NC * NS
RPW = B // NW
CH = 32
NCHUNK = RPW // CH

C_ITEMH = 0
C_CATEH = 56
C_SING = 112
LP = 56


def _sc_hist_body(ids_cc, t_hi, t_hc, gath, idxb, hbuf, obuf, sem):
    wid = lax.axis_index("s") * NC + lax.axis_index("c")
    row0 = wid * RPW
    iota = lax.iota(jnp.int32, 16)
    zeros = jnp.zeros((16,), jnp.float32)

    def chunk_body(ch, _):
        base = row0 + ch * CH
        pltpu.async_copy(ids_cc.at[pl.ds(base, CH)], idxb, sem).wait()
        dmas = []
        for j in range(CH):
            dmas.append(pltpu.async_copy(
                t_hi.at[idxb.at[j, pl.ds(C_ITEMH, LP)]],
                hbuf.at[pl.ds(j * LP, LP)], sem))
        for j in range(CH):
            dmas.append(pltpu.async_copy(
                t_hc.at[idxb.at[j, pl.ds(C_CATEH, LP)]],
                hbuf.at[pl.ds(CH * LP + j * LP, LP)], sem))
        for hnd in dmas:
            hnd.wait()

        def row_body(r, _):
            fr = iota * 0 + r
            acc_i = zeros
            for j in range(L):
                acc_i = acc_i + hbuf[r * LP + j, :]
            plsc.store_scatter(obuf, [fr, iota], acc_i * (1.0 / L))
            acc_c = zeros
            for j in range(L):
                acc_c = acc_c + hbuf[CH * LP + r * LP + j, :]
            plsc.store_scatter(obuf, [fr, 16 + iota], acc_c * (1.0 / L))
            return 0

        lax.fori_loop(0, CH, row_body, 0)
        pltpu.sync_copy(obuf, gath.at[pl.ds(base, CH)])
        return 0

    lax.fori_loop(0, NCHUNK, chunk_body, 0)


@functools.partial(
    pl.kernel,
    out_type=jax.ShapeDtypeStruct((B, 32), jnp.float32),
    mesh=plsc.VectorSubcoreMesh(core_axis_name="c", subcore_axis_name="s",
                                num_cores=NC),
    compiler_params=pltpu.CompilerParams(needs_layout_passes=False,
                                         use_tc_tiling_on_sc=False),
    scratch_types=[
        pltpu.VMEM((CH, 128), jnp.int32),
        pltpu.VMEM((2 * CH * LP, D), jnp.float32),
        pltpu.VMEM((CH, 32), jnp.float32),
        pltpu.SemaphoreType.DMA,
    ],
)
def _sc_hist_kernel(*args):
    _sc_hist_body(*args)


SB = 8


def _sc_single_body(ids_cc, t_u, t_i, t_c, t_s, gath, idxb, bbuf, obuf, sem):
    wid = lax.axis_index("s") * NC + lax.axis_index("c")
    row0 = wid * RPW
    iota = lax.iota(jnp.int32, 16)
    singles = (t_u, t_i, t_c, t_s)

    def chunk_body(ch, _):
        base = row0 + ch * CH
        pltpu.async_copy(ids_cc.at[pl.ds(base, CH)], idxb, sem).wait()
        for sb in range(CH // SB):
            dmas = []
            cols = []
            for j8 in range(SB):
                j = sb * SB + j8
                sv = idxb[j, pl.ds(C_SING, 16)]
                for t in range(4):
                    idx = sv[t]
                    blk = pl.multiple_of((idx // 128) * 128, 128)
                    cols.append(idx - blk)
                    k = j8 * 4 + t
                    dmas.append(pltpu.async_copy(
                        singles[t].at[:, pl.ds(blk, 128)],
                        bbuf.at[pl.ds(k * 16, 16)], sem))
            for hnd in dmas:
                hnd.wait()
            for j8 in range(SB):
                j = sb * SB + j8
                fr = iota * 0 + j
                for t in range(4):
                    k = j8 * 4 + t
                    col = cols[k]
                    v = plsc.load_gather(bbuf, [k * 16 + iota,
                                                iota * 0 + col])
                    plsc.store_scatter(obuf, [fr, t * 16 + iota], v)
        pltpu.sync_copy(obuf, gath.at[pl.ds(base, CH)])
        return 0

    lax.fori_loop(0, NCHUNK, chunk_body, 0)


@functools.partial(
    pl.kernel,
    out_type=jax.ShapeDtypeStruct((B, 128), jnp.float32),
    mesh=plsc.VectorSubcoreMesh(core_axis_name="c", subcore_axis_name="s",
                                num_cores=NC),
    compiler_params=pltpu.CompilerParams(needs_layout_passes=False,
                                         use_tc_tiling_on_sc=True),
    scratch_types=[
        pltpu.VMEM((CH, 128), jnp.int32),
        pltpu.VMEM((4 * SB * 16, 128), jnp.float32),
        pltpu.VMEM((CH, 128), jnp.float32),
        pltpu.SemaphoreType.DMA,
    ],
)
def _sc_single_kernel(*args):
    _sc_single_body(*args)


RB = 1024


def _tc_body(sing_ref, hist_ref, price_ref, ctr_ref, hp_ref, cst_ref,
             out_ref):
    ps = cst_ref[0, 0]
    pb = cst_ref[1, 0]
    wce = cst_ref[2:3, :]
    bce = cst_ref[3:4, :]
    whp = cst_ref[4:5, :]
    bhp = cst_ref[5:6, :]
    out_ref[:, 0:1] = price_ref[:, :] * ps + pb
    out_ref[:, 1:17] = ctr_ref[:, :] * wce + bce
    out_ref[:, 17:81] = sing_ref[:, 0:64]
    s = jnp.sum(hp_ref[:, :], axis=1, keepdims=True)
    out_ref[:, 81:97] = s * whp + bhp
    out_ref[:, 97:129] = hist_ref[:, :]


_tc_kernel = pl.pallas_call(
    _tc_body,
    out_shape=jax.ShapeDtypeStruct((B, 129), jnp.float32),
    grid=(B // RB,),
    in_specs=[
        pl.BlockSpec((RB, 128), lambda i: (i, 0)),
        pl.BlockSpec((RB, 32), lambda i: (i, 0)),
        pl.BlockSpec((RB, 1), lambda i: (i, 0)),
        pl.BlockSpec((RB, 1), lambda i: (i, 0)),
        pl.BlockSpec((RB, L), lambda i: (i, 0)),
        pl.BlockSpec((6, 16), lambda i: (0, 0)),
    ],
    out_specs=pl.BlockSpec((RB, 129), lambda i: (i, 0)),
)


def kernel(price, ctr, user_id, item_id, cate_id, shop_id, hist_item_id,
           hist_cate_id, hist_price, T_user, T_item, T_cate, T_shop,
           T_hist_item, T_hist_cate, W_ctr, W_hp,
           g_price, b_price, g_ctr, b_ctr, g_hp, b_hp):
    rs = 1.0 / jnp.sqrt(jnp.float32(1.0 + EPS))
    consts = jnp.stack([
        jnp.broadcast_to(g_price[0] * rs, (16,)),
        jnp.broadcast_to(b_price[0], (16,)),
        (g_ctr[0] * rs) * W_ctr[0],
        b_ctr[0] * W_ctr[0],
        (g_hp[0] * rs / L) * W_hp[0],
        b_hp[0] * W_hp[0],
    ]).astype(jnp.float32)
    zi6 = jnp.zeros((B, 6), jnp.int32)
    ids_cc = jnp.concatenate(
        [hist_item_id, zi6, hist_cate_id, zi6,
         user_id, item_id, cate_id, shop_id,
         jnp.zeros((B, 12), jnp.int32)], axis=1)
    sing = _sc_single_kernel(ids_cc, T_user.T, T_item.T, T_cate.T, T_shop.T)
    hist = _sc_hist_kernel(ids_cc, T_hist_item, T_hist_cate)
    return _tc_kernel(sing, hist, price, ctr, hist_price, consts)


# speedup vs baseline: 1.5644x; 1.1567x over previous
import functools

import jax
import jax.numpy as jnp
from jax import lax
from jax.experimental import pallas as pl
from jax.experimental.pallas import tpu as pltpu
from jax.experimental.pallas import tpu_sc as plsc

B = 16384
L = 50
D = 16
EPS = 1e-3

NC = 2
NS = 16
NW = NC * NS
RPW = B // NW
CH = 32
NCHUNK = RPW // CH

C_ITEMH = 0
C_CATEH = 56
C_SING = 112
LP = 56


def _sc_hist_body(ids_cc, t_hi, t_hc, dep, gath, idxb, hbuf, obuf, sem):
    del dep
    wid = lax.axis_index("s") * NC + lax.axis_index("c")
    row0 = wid * RPW
    iota = lax.iota(jnp.int32, 16)
    zeros = jnp.zeros((16,), jnp.float32)

    def chunk_body(ch, _):
        base = row0 + ch * CH
        pltpu.async_copy(ids_cc.at[pl.ds(base, CH)], idxb, sem).wait()
        dmas = []
        for j in range(CH):
            dmas.append(pltpu.async_copy(
                t_hi.at[idxb.at[j, pl.ds(C_ITEMH, LP)]],
                hbuf.at[pl.ds(j * LP, LP)], sem))
        for j in range(CH):
            dmas.append(pltpu.async_copy(
                t_hc.at[idxb.at[j, pl.ds(C_CATEH, LP)]],
                hbuf.at[pl.ds(CH * LP + j * LP, LP)], sem))
        for hnd in dmas:
            hnd.wait()

        def row_body(r, _):
            fr = iota * 0 + r
            acc_i = zeros
            for j in range(L):
                acc_i = acc_i + hbuf[r * LP + j, :]
            plsc.store_scatter(obuf, [fr, iota], acc_i * (1.0 / L))
            acc_c = zeros
            for j in range(L):
                acc_c = acc_c + hbuf[CH * LP + r * LP + j, :]
            plsc.store_scatter(obuf, [fr, 16 + iota], acc_c * (1.0 / L))
            return 0

        lax.fori_loop(0, CH, row_body, 0)
        pltpu.sync_copy(obuf, gath.at[pl.ds(base, CH)])
        return 0

    lax.fori_loop(0, NCHUNK, chunk_body, 0)


@functools.partial(
    pl.kernel,
    out_type=jax.ShapeDtypeStruct((B, 32), jnp.float32),
    mesh=plsc.VectorSubcoreMesh(core_axis_name="c", subcore_axis_name="s",
                                num_cores=NC),
    compiler_params=pltpu.CompilerParams(needs_layout_passes=False,
                                         use_tc_tiling_on_sc=False),
    scratch_types=[
        pltpu.VMEM((CH, 128), jnp.int32),
        pltpu.VMEM((2 * CH * LP, D), jnp.float32),
        pltpu.VMEM((CH, 32), jnp.float32),
        pltpu.SemaphoreType.DMA,
    ],
)
def _sc_hist_kernel(*args):
    _sc_hist_body(*args)


SB = 8


def _sc_single_body(ids_cc, t_u, t_i, t_c, t_s, gath, idxb, bbuf, obuf, sem):
    wid = lax.axis_index("s") * NC + lax.axis_index("c")
    row0 = wid * RPW
    iota = lax.iota(jnp.int32, 16)
    singles = (t_u, t_i, t_c, t_s)

    def chunk_body(ch, _):
        base = row0 + ch * CH
        pltpu.async_copy(ids_cc.at[pl.ds(base, CH)], idxb, sem).wait()
        for sb in range(CH // SB):
            dmas = []
            cols = []
            for j8 in range(SB):
                j = sb * SB + j8
                sv = idxb[j, pl.ds(C_SING, 16)]
                for t in range(4):
                    idx = sv[t]
                    blk = pl.multiple_of((idx // 128) * 128, 128)
                    cols.append(idx - blk)
                    k = j8 * 4 + t
                    dmas.append(pltpu.async_copy(
                        singles[t].at[:, pl.ds(blk, 128)],
                        bbuf.at[pl.ds(k * 16, 16)], sem))
            for hnd in dmas:
                hnd.wait()
            for j8 in range(SB):
                j = sb * SB + j8
                fr = iota * 0 + j
                for t in range(4):
                    k = j8 * 4 + t
                    col = cols[k]
                    v = plsc.load_gather(bbuf, [k * 16 + iota,
                                                iota * 0 + col])
                    plsc.store_scatter(obuf, [fr, t * 16 + iota], v)
        pltpu.sync_copy(obuf, gath.at[pl.ds(base, CH)])
        return 0

    lax.fori_loop(0, NCHUNK, chunk_body, 0)


@functools.partial(
    pl.kernel,
    out_type=jax.ShapeDtypeStruct((B, 128), jnp.float32),
    mesh=plsc.VectorSubcoreMesh(core_axis_name="c", subcore_axis_name="s",
                                num_cores=NC),
    compiler_params=pltpu.CompilerParams(needs_layout_passes=False,
                                         use_tc_tiling_on_sc=True),
    scratch_types=[
        pltpu.VMEM((CH, 128), jnp.int32),
        pltpu.VMEM((4 * SB * 16, 128), jnp.float32),
        pltpu.VMEM((CH, 128), jnp.float32),
        pltpu.SemaphoreType.DMA,
    ],
)
def _sc_single_kernel(*args):
    _sc_single_body(*args)


RB = 1024


def _tc_body(sing_ref, hist_ref, price_ref, ctr_ref, hp_ref, cst_ref,
             out_ref):
    ps = cst_ref[0, 0]
    pb = cst_ref[1, 0]
    wce = cst_ref[2:3, :]
    bce = cst_ref[3:4, :]
    whp = cst_ref[4:5, :]
    bhp = cst_ref[5:6, :]
    out_ref[:, 0:1] = price_ref[:, :] * ps + pb
    out_ref[:, 1:17] = ctr_ref[:, :] * wce + bce
    out_ref[:, 17:81] = sing_ref[:, 0:64]
    s = jnp.sum(hp_ref[:, :], axis=1, keepdims=True)
    out_ref[:, 81:97] = s * whp + bhp
    out_ref[:, 97:129] = hist_ref[:, :]


_tc_kernel = pl.pallas_call(
    _tc_body,
    out_shape=jax.ShapeDtypeStruct((B, 129), jnp.float32),
    grid=(B // RB,),
    in_specs=[
        pl.BlockSpec((RB, 128), lambda i: (i, 0)),
        pl.BlockSpec((RB, 32), lambda i: (i, 0)),
        pl.BlockSpec((RB, 1), lambda i: (i, 0)),
        pl.BlockSpec((RB, 1), lambda i: (i, 0)),
        pl.BlockSpec((RB, L), lambda i: (i, 0)),
        pl.BlockSpec((6, 16), lambda i: (0, 0)),
    ],
    out_specs=pl.BlockSpec((RB, 129), lambda i: (i, 0)),
)


def kernel(price, ctr, user_id, item_id, cate_id, shop_id, hist_item_id,
           hist_cate_id, hist_price, T_user, T_item, T_cate, T_shop,
           T_hist_item, T_hist_cate, W_ctr, W_hp,
           g_price, b_price, g_ctr, b_ctr, g_hp, b_hp):
    rs = 1.0 / jnp.sqrt(jnp.float32(1.0 + EPS))
    consts = jnp.stack([
        jnp.broadcast_to(g_price[0] * rs, (16,)),
        jnp.broadcast_to(b_price[0], (16,)),
        (g_ctr[0] * rs) * W_ctr[0],
        b_ctr[0] * W_ctr[0],
        (g_hp[0] * rs / L) * W_hp[0],
        b_hp[0] * W_hp[0],
    ]).astype(jnp.float32)
    zi6 = jnp.zeros((B, 6), jnp.int32)
    ids_cc = jnp.concatenate(
        [hist_item_id, zi6, hist_cate_id, zi6,
         user_id, item_id, cate_id, shop_id,
         jnp.zeros((B, 12), jnp.int32)], axis=1)
    sing = _sc_single_kernel(ids_cc, T_user.T, T_item.T, T_cate.T, T_shop.T)
    hist = _sc_hist_kernel(ids_cc, T_hist_item, T_hist_cate, sing)
    return _tc_kernel(sing, hist, price, ctr, hist_price, consts)
